# baseline (speedup 1.0000x reference)
import functools

import jax
import jax.numpy as jnp
from jax import lax
from jax.experimental import pallas as pl
from jax.experimental.pallas import tpu as pltpu

_BN = 2048


def _body(surf_ref, emb_ref, w_ref, b_ref, o_ref, fhs_ref, *, T):
    @pl.when(pl.program_id(0) == 0)
    def _():
        s = surf_ref[...]
        B, T_ = s.shape
        V = emb_ref.shape[0]
        oh = (s[:, :, None] == lax.broadcasted_iota(jnp.int32, (B, T_, V), 2))
        counts = jnp.sum(oh.astype(jnp.float32), axis=1)
        fhs_ref[...] = jnp.dot(
            counts, emb_ref[...], preferred_element_type=jnp.float32) * (1.0 / T)

    o_ref[...] = (
        jnp.dot(fhs_ref[...], w_ref[...], preferred_element_type=jnp.float32)
        + b_ref[...])


def kernel(surf, char_emb, root_codebook, suffix_W, suffix_b, suffix_codebook,
           ord_W, ord_b, ord_codebooks, out_W, out_b):
    B, T = surf.shape
    V, D = char_emb.shape
    _, N = out_W.shape
    nb = (N + _BN - 1) // _BN
    b2d = out_b.reshape(1, N)

    out2d = pl.pallas_call(
        functools.partial(_body, T=T),
        grid=(nb,),
        in_specs=[
            pl.BlockSpec((B, T), lambda i: (0, 0)),
            pl.BlockSpec((V, D), lambda i: (0, 0)),
            pl.BlockSpec((D, _BN), lambda i: (0, i)),
            pl.BlockSpec((1, _BN), lambda i: (0, i)),
        ],
        out_specs=pl.BlockSpec((B, _BN), lambda i: (0, i)),
        out_shape=jax.ShapeDtypeStruct((B, N), jnp.float32),
        scratch_shapes=[pltpu.VMEM((B, D), jnp.float32)],
        compiler_params=pltpu.CompilerParams(
            dimension_semantics=("arbitrary",)),
    )(surf, char_emb, out_W, b2d)
    return out2d[:, None, :]


# speedup vs baseline: 1.0011x; 1.0011x over previous
import jax
import jax.numpy as jnp
from jax import lax
from jax.experimental import pallas as pl
from jax.experimental.pallas import tpu as pltpu

_BN = 2048


def _fhs_body(surf_ref, emb_ref, o_ref):
    s = surf_ref[...]
    B, T = s.shape
    V = emb_ref.shape[0]
    oh = (s[:, :, None] == lax.broadcasted_iota(jnp.int32, (B, T, V), 2))
    counts = jnp.sum(oh.astype(jnp.float32), axis=1)
    o_ref[...] = jnp.dot(
        counts, emb_ref[...], preferred_element_type=jnp.float32) * (1.0 / T)


def _proj_body(fhs_ref, w_ref, b_ref, o_ref):
    o_ref[...] = (
        jnp.dot(fhs_ref[...], w_ref[...], preferred_element_type=jnp.float32)
        + b_ref[...])


def kernel(surf, char_emb, root_codebook, suffix_W, suffix_b, suffix_codebook,
           ord_W, ord_b, ord_codebooks, out_W, out_b):
    B, T = surf.shape
    V, D = char_emb.shape
    _, N = out_W.shape
    nb = (N + _BN - 1) // _BN
    b2d = out_b.reshape(1, N)

    fhs = pl.pallas_call(
        _fhs_body,
        out_shape=jax.ShapeDtypeStruct((B, D), jnp.float32),
    )(surf, char_emb)

    out2d = pl.pallas_call(
        _proj_body,
        grid=(nb,),
        in_specs=[
            pl.BlockSpec((B, D), lambda i: (0, 0)),
            pl.BlockSpec((D, _BN), lambda i: (0, i)),
            pl.BlockSpec((1, _BN), lambda i: (0, i)),
        ],
        out_specs=pl.BlockSpec((B, _BN), lambda i: (0, i)),
        out_shape=jax.ShapeDtypeStruct((B, N), jnp.float32),
        compiler_params=pltpu.CompilerParams(
            dimension_semantics=("parallel",)),
    )(fhs, out_W, b2d)
    return out2d[:, None, :]
